# baseline (speedup 1.0000x reference)
import functools

import jax
import jax.numpy as jnp
from jax import lax
from jax.experimental import pallas as pl
from jax.experimental.pallas import tpu as pltpu
from jax.experimental.pallas import tpu_sc as plsc

NC = 2
NS = 16
NW = NC * NS
CH = 80
NP = 10240


def _fill_2d(ref, nrows, ncols, value):
    v = jnp.full((16,), value, jnp.float32)
    npc = ncols // 16

    def body(i, _):
        r = i // npc
        c = (i % npc) * 16
        ref[r, pl.ds(c, 16)] = v
        return 0

    lax.fori_loop(0, nrows * npc, body, 0)


def _fill_1d(ref, n, value):
    v = jnp.full((16,), value, jnp.float32)

    def body(i, _):
        ref[pl.ds(i * 16, 16)] = v
        return 0

    lax.fori_loop(0, n // 16, body, 0)


@functools.lru_cache(maxsize=None)
def _make_deg_kernel(n, e):
    epw = e // NW
    nch = epw // CH
    mesh = plsc.VectorSubcoreMesh(core_axis_name="c", subcore_axis_name="s")
    out1 = jax.ShapeDtypeStruct((n,), jnp.float32)

    @functools.partial(
        pl.kernel,
        out_type=(out1, out1, out1, out1),
        mesh=mesh,
        scratch_types=[
            pltpu.VMEM((n,), jnp.float32),
            pltpu.VMEM((CH,), jnp.float32),
            pltpu.VMEM((10, 2, CH), jnp.int32),
            pltpu.VMEM_SHARED((n,), jnp.float32),
            pltpu.VMEM_SHARED((n,), jnp.float32),
            pltpu.SemaphoreType.DMA,
            pltpu.SemaphoreType.DMA,
        ],
    )
    def k(ei_hbm, o_c0, o_c1, i_c0, i_c1,
          zbuf, ones, idx, acc_o, acc_i, sem_i, sem_s):
        cid = lax.axis_index("c")
        sid = lax.axis_index("s")
        wid = sid * NC + cid

        def idx_load(t):
            pltpu.async_copy(ei_hbm.at[wid, t], idx.at[t % 10], sem_i)

        def wait_idx():
            pltpu.make_async_copy(ei_hbm.at[0, 0], idx.at[0], sem_i).wait()

        def wait_scatter():
            pltpu.make_async_copy(ones, acc_o.at[idx.at[0, 0]], sem_s).wait()

        for t in range(4):
            idx_load(t)
        _fill_1d(ones, CH, 1.0)

        @pl.when(sid == 0)
        def _():
            _fill_1d(zbuf, n, 0.0)
            pltpu.sync_copy(zbuf, acc_o)
            pltpu.sync_copy(zbuf, acc_i)

        plsc.subcore_barrier()

        def body(t, _):
            wait_idx()
            pltpu.async_copy(ones, acc_o.at[idx.at[t % 10, 0]], sem_s, add=True)
            pltpu.async_copy(ones, acc_i.at[idx.at[t % 10, 1]], sem_s, add=True)

            @pl.when(t >= 4)
            def _():
                wait_scatter()
                wait_scatter()

            @pl.when(t + 4 < nch)
            def _():
                idx_load(t + 4)

            return 0

        lax.fori_loop(0, nch, body, 0)
        for _ in range(8):
            wait_scatter()
        plsc.subcore_barrier()

        @pl.when(jnp.logical_and(sid == 0, cid == 0))
        def _():
            pltpu.sync_copy(acc_o, o_c0)
            pltpu.sync_copy(acc_i, i_c0)

        @pl.when(jnp.logical_and(sid == 0, cid == 1))
        def _():
            pltpu.sync_copy(acc_o, o_c1)
            pltpu.sync_copy(acc_i, i_c1)

    return k


@functools.lru_cache(maxsize=None)
def _make_edge_kernel(n, e, d):
    epw = e // NW
    nch = epw // CH
    rpt = NP // NS
    zr = 32
    g_depth = 2 if d >= 128 else 5
    rr = 2 * g_depth
    ri = 2 * g_depth + 2
    mesh = plsc.VectorSubcoreMesh(core_axis_name="c", subcore_axis_name="s")

    @functools.partial(
        pl.kernel,
        out_type=jax.ShapeDtypeStruct((NC, NP, d), jnp.float32),
        mesh=mesh,
        scratch_types=[
            pltpu.VMEM((ri, 2, CH), jnp.int32),
            pltpu.VMEM((rr, CH, d), jnp.float32),
            pltpu.VMEM((zr, d), jnp.float32),
            pltpu.VMEM_SHARED((NP, d), jnp.float32),
            pltpu.SemaphoreType.DMA,
            pltpu.SemaphoreType.DMA,
            pltpu.SemaphoreType.DMA,
        ],
        compiler_params=pltpu.CompilerParams(use_tc_tiling_on_sc=(d % 128 == 0)),
    )
    def k(g_hbm, ei_hbm, out, idx, rows, zbuf, acc, sem_i, sem_g, sem_s):
        cid = lax.axis_index("c")
        sid = lax.axis_index("s")
        wid = sid * NC + cid

        def idx_load(t):
            pltpu.async_copy(ei_hbm.at[wid, t], idx.at[t % ri], sem_i)

        def gather(t):
            pltpu.async_copy(g_hbm.at[idx.at[t % ri, 0]], rows.at[t % rr], sem_g)

        def scatter(t):
            pltpu.async_copy(rows.at[t % rr], acc.at[idx.at[t % ri, 1]], sem_s,
                             add=True)

        def wait_idx():
            pltpu.make_async_copy(ei_hbm.at[0, 0], idx.at[0], sem_i).wait()

        def wait_gather():
            pltpu.make_async_copy(g_hbm.at[idx.at[0, 0]], rows.at[0], sem_g).wait()

        def wait_scatter():
            pltpu.make_async_copy(rows.at[0], acc.at[idx.at[0, 1]], sem_s).wait()

        for t in range(g_depth + 1):
            idx_load(t)
        _fill_2d(zbuf, zr, d, 0.0)
        r0 = sid * rpt
        for j in range(rpt // zr):
            pltpu.async_copy(zbuf, acc.at[pl.ds(r0 + j * zr, zr)], sem_s)
        for j in range(rpt // zr):
            pltpu.make_async_copy(zbuf, acc.at[pl.ds(r0, zr)], sem_s).wait()
        plsc.subcore_barrier()

        for t in range(g_depth):
            wait_idx()
            gather(t)

        def body(t, _):
            wait_gather()
            scatter(t)

            @pl.when(t + g_depth < nch)
            def _():
                wait_idx()

                @pl.when(t >= g_depth)
                def _():
                    wait_scatter()

                gather(t + g_depth)

                @pl.when(t + g_depth + 1 < nch)
                def _():
                    idx_load(t + g_depth + 1)

            return 0

        lax.fori_loop(0, nch, body, 0)
        for _ in range(2 * g_depth):
            wait_scatter()
        plsc.subcore_barrier()
        pltpu.sync_copy(acc.at[pl.ds(r0, rpt)], out.at[cid, pl.ds(r0, rpt)])

    return k


def _rs_g1(o0, o1, i0, i1, x, w):
    n, kdim = x.shape
    d2 = w.shape[1]
    bn = 2000

    def body(o0_ref, o1_ref, i0_ref, i1_ref, x_ref, w_ref,
             g_ref, ro_ref, ri_ref):
        ro = lax.rsqrt(jnp.maximum(o0_ref[...] + o1_ref[...], 1.0))
        ro_ref[...] = ro
        ri_ref[...] = lax.rsqrt(jnp.maximum(i0_ref[...] + i1_ref[...], 1.0))
        g_ref[...] = jnp.dot(x_ref[...] * ro, w_ref[...],
                             preferred_element_type=jnp.float32)

    vec = pl.BlockSpec((bn, 1), lambda i: (i, 0))
    return pl.pallas_call(
        body,
        grid=(n // bn,),
        in_specs=[
            vec, vec, vec, vec,
            pl.BlockSpec((bn, kdim), lambda i: (i, 0)),
            pl.BlockSpec((kdim, d2), lambda i: (0, 0)),
        ],
        out_specs=[pl.BlockSpec((bn, d2), lambda i: (i, 0)), vec, vec],
        out_shape=[
            jax.ShapeDtypeStruct((n, d2), jnp.float32),
            jax.ShapeDtypeStruct((n, 1), jnp.float32),
            jax.ShapeDtypeStruct((n, 1), jnp.float32),
        ],
    )(o0, o1, i0, i1, x, w)


def _layer_mid(s, rs_in, b, rs_out, w):
    kdim = s.shape[2]
    n = rs_in.shape[0]
    d2 = w.shape[1]
    bn = 2000

    def body(s_ref, ri_ref, b_ref, ro_ref, w_ref, o_ref):
        h = jax.nn.relu((s_ref[0] + s_ref[1]) * ri_ref[...] + b_ref[...])
        o_ref[...] = jnp.dot(h * ro_ref[...], w_ref[...],
                             preferred_element_type=jnp.float32)

    return pl.pallas_call(
        body,
        grid=(n // bn,),
        in_specs=[
            pl.BlockSpec((2, bn, kdim), lambda i: (0, i, 0)),
            pl.BlockSpec((bn, 1), lambda i: (i, 0)),
            pl.BlockSpec((1, kdim), lambda i: (0, 0)),
            pl.BlockSpec((bn, 1), lambda i: (i, 0)),
            pl.BlockSpec((kdim, d2), lambda i: (0, 0)),
        ],
        out_specs=pl.BlockSpec((bn, d2), lambda i: (i, 0)),
        out_shape=jax.ShapeDtypeStruct((n, d2), jnp.float32),
    )(s, rs_in, b, rs_out, w)


def _final(s, rs_in, b):
    d2 = s.shape[2]
    n = rs_in.shape[0]
    bn = 2000

    def body(s_ref, ri_ref, b_ref, o_ref):
        o_ref[...] = (s_ref[0] + s_ref[1]) * ri_ref[...] + b_ref[...]

    return pl.pallas_call(
        body,
        grid=(n // bn,),
        in_specs=[
            pl.BlockSpec((2, bn, d2), lambda i: (0, i, 0)),
            pl.BlockSpec((bn, 1), lambda i: (i, 0)),
            pl.BlockSpec((1, d2), lambda i: (0, 0)),
        ],
        out_specs=pl.BlockSpec((bn, d2), lambda i: (i, 0)),
        out_shape=jax.ShapeDtypeStruct((n, d2), jnp.float32),
    )(s, rs_in, b)


def kernel(x, edge_index, W0, b0, W1, b1, W2, b2):
    n, d_in = x.shape
    e = edge_index.shape[1]
    d_h = W0.shape[1]
    n_cls = W2.shape[1]
    nch = e // NW // CH

    ei4 = jnp.transpose(edge_index.reshape(2, NW, nch, CH), (1, 2, 0, 3))

    o_c0, o_c1, i_c0, i_c1 = _make_deg_kernel(n, e)(ei4)

    edge128 = _make_edge_kernel(n, e, d_h)
    edge_cls = _make_edge_kernel(n, e, n_cls)

    g, rs_out, rs_in = _rs_g1(o_c0.reshape(n, 1), o_c1.reshape(n, 1),
                              i_c0.reshape(n, 1), i_c1.reshape(n, 1), x, W0)
    s = edge128(g, ei4)
    g = _layer_mid(s, rs_in, b0.reshape(1, d_h), rs_out, W1)
    s = edge128(g, ei4)
    g = _layer_mid(s, rs_in, b1.reshape(1, d_h), rs_out, W2)
    s = edge_cls(g, ei4)
    return _final(s, rs_in, b2.reshape(1, n_cls))


# speedup vs baseline: 1.0578x; 1.0578x over previous
import functools

import jax
import jax.numpy as jnp
from jax import lax
from jax.experimental import pallas as pl
from jax.experimental.pallas import tpu as pltpu
from jax.experimental.pallas import tpu_sc as plsc

NC = 2
NS = 16
NW = NC * NS
CH = 80
NP = 10240


def _fill_2d(ref, nrows, ncols, value):
    v = jnp.full((16,), value, jnp.float32)
    npc = ncols // 16

    def body(i, _):
        r = i // npc
        c = (i % npc) * 16
        ref[r, pl.ds(c, 16)] = v
        return 0

    lax.fori_loop(0, nrows * npc, body, 0)


def _fill_1d(ref, n, value):
    v = jnp.full((16,), value, jnp.float32)

    def body(i, _):
        ref[pl.ds(i * 16, 16)] = v
        return 0

    lax.fori_loop(0, n // 16, body, 0)


@functools.lru_cache(maxsize=None)
def _make_deg_kernel(n, e):
    epw = e // NW
    nch = epw // CH
    mesh = plsc.VectorSubcoreMesh(core_axis_name="c", subcore_axis_name="s")
    out1 = jax.ShapeDtypeStruct((n,), jnp.float32)

    @functools.partial(
        pl.kernel,
        out_type=(out1, out1, out1, out1),
        mesh=mesh,
        scratch_types=[
            pltpu.VMEM((n,), jnp.float32),
            pltpu.VMEM((CH,), jnp.float32),
            pltpu.VMEM((10, 2, CH), jnp.int32),
            pltpu.VMEM_SHARED((n,), jnp.float32),
            pltpu.VMEM_SHARED((n,), jnp.float32),
            pltpu.SemaphoreType.DMA,
            pltpu.SemaphoreType.DMA,
        ],
    )
    def k(ei_hbm, o_c0, o_c1, i_c0, i_c1,
          zbuf, ones, idx, acc_o, acc_i, sem_i, sem_s):
        cid = lax.axis_index("c")
        sid = lax.axis_index("s")
        wid = sid * NC + cid

        def idx_load(t):
            pltpu.async_copy(ei_hbm.at[wid, t], idx.at[t % 10], sem_i)

        def wait_idx():
            pltpu.make_async_copy(ei_hbm.at[0, 0], idx.at[0], sem_i).wait()

        def wait_scatter():
            pltpu.make_async_copy(ones, acc_o.at[idx.at[0, 0]], sem_s).wait()

        for t in range(4):
            idx_load(t)
        _fill_1d(ones, CH, 1.0)

        @pl.when(sid == 0)
        def _():
            _fill_1d(zbuf, n, 0.0)
            pltpu.sync_copy(zbuf, acc_o)
            pltpu.sync_copy(zbuf, acc_i)

        plsc.subcore_barrier()

        def body(t, _):
            wait_idx()
            pltpu.async_copy(ones, acc_o.at[idx.at[t % 10, 0]], sem_s, add=True)
            pltpu.async_copy(ones, acc_i.at[idx.at[t % 10, 1]], sem_s, add=True)

            @pl.when(t >= 4)
            def _():
                wait_scatter()
                wait_scatter()

            @pl.when(t + 4 < nch)
            def _():
                idx_load(t + 4)

            return 0

        lax.fori_loop(0, nch, body, 0)
        for _ in range(8):
            wait_scatter()
        plsc.subcore_barrier()

        @pl.when(jnp.logical_and(sid == 0, cid == 0))
        def _():
            pltpu.sync_copy(acc_o, o_c0)
            pltpu.sync_copy(acc_i, i_c0)

        @pl.when(jnp.logical_and(sid == 0, cid == 1))
        def _():
            pltpu.sync_copy(acc_o, o_c1)
            pltpu.sync_copy(acc_i, i_c1)

    return k


@functools.lru_cache(maxsize=None)
def _make_edge_kernel(n, e, d):
    epw = e // NW
    nch = epw // CH
    rpt = NP // NS
    zr = 32
    g_depth = 2 if d >= 128 else 5
    rr = 2 * g_depth
    ri = 2 * g_depth + 2
    mesh = plsc.VectorSubcoreMesh(core_axis_name="c", subcore_axis_name="s")

    @functools.partial(
        pl.kernel,
        out_type=jax.ShapeDtypeStruct((NC, NP, d), jnp.float32),
        mesh=mesh,
        scratch_types=[
            pltpu.VMEM((ri, 2, CH), jnp.int32),
            pltpu.VMEM((rr, CH, d), jnp.float32),
            pltpu.VMEM((zr, d), jnp.float32),
            pltpu.VMEM_SHARED((NP, d), jnp.float32),
            pltpu.SemaphoreType.DMA,
            pltpu.SemaphoreType.DMA,
            pltpu.SemaphoreType.DMA,
        ],
        compiler_params=pltpu.CompilerParams(use_tc_tiling_on_sc=(d % 128 == 0)),
    )
    def k(g_hbm, ei_hbm, out, idx, rows, zbuf, acc, sem_i, sem_g, sem_s):
        cid = lax.axis_index("c")
        sid = lax.axis_index("s")
        wid = sid * NC + cid

        def idx_load(t):
            pltpu.async_copy(ei_hbm.at[wid, t], idx.at[t % ri], sem_i)

        def gather(t):
            pltpu.async_copy(g_hbm.at[idx.at[t % ri, 0]], rows.at[t % rr], sem_g)

        def scatter(t):
            pltpu.async_copy(rows.at[t % rr], acc.at[idx.at[t % ri, 1]], sem_s,
                             add=True)

        def wait_idx():
            pltpu.make_async_copy(ei_hbm.at[0, 0], idx.at[0], sem_i).wait()

        def wait_gather():
            pltpu.make_async_copy(g_hbm.at[idx.at[0, 0]], rows.at[0], sem_g).wait()

        def wait_scatter():
            pltpu.make_async_copy(rows.at[0], acc.at[idx.at[0, 1]], sem_s).wait()

        for t in range(g_depth + 1):
            idx_load(t)
        _fill_2d(zbuf, zr, d, 0.0)
        r0 = sid * rpt
        for j in range(rpt // zr):
            pltpu.async_copy(zbuf, acc.at[pl.ds(r0 + j * zr, zr)], sem_s)
        for j in range(rpt // zr):
            pltpu.make_async_copy(zbuf, acc.at[pl.ds(r0, zr)], sem_s).wait()
        plsc.subcore_barrier()

        for t in range(g_depth):
            wait_idx()
            gather(t)

        def body(t, _):
            wait_gather()
            scatter(t)

            @pl.when(t + g_depth < nch)
            def _():
                wait_idx()

                @pl.when(t >= g_depth)
                def _():
                    wait_scatter()

                gather(t + g_depth)

                @pl.when(t + g_depth + 1 < nch)
                def _():
                    idx_load(t + g_depth + 1)

            return 0

        lax.fori_loop(0, nch, body, 0)
        for _ in range(2 * g_depth):
            wait_scatter()
        plsc.subcore_barrier()
        pltpu.sync_copy(acc.at[pl.ds(r0, rpt)], out.at[cid, pl.ds(r0, rpt)])

    return k


def _rs_from_deg(o_c0, o_c1, i_c0, i_c1):
    def body(a_ref, b_ref, c_ref, d_ref, o_ref):
        o_ref[0, :] = lax.rsqrt(jnp.maximum(a_ref[...] + b_ref[...], 1.0))
        o_ref[1, :] = lax.rsqrt(jnp.maximum(c_ref[...] + d_ref[...], 1.0))

    return pl.pallas_call(
        body,
        out_shape=jax.ShapeDtypeStruct((2, o_c0.shape[0]), jnp.float32),
    )(o_c0, o_c1, i_c0, i_c1)


def _scale_mm(x, rs, w):
    n, kdim = x.shape
    d2 = w.shape[1]
    bn = 2000

    def body(x_ref, rs_ref, w_ref, o_ref):
        o_ref[...] = jnp.dot(x_ref[...] * rs_ref[...], w_ref[...],
                             preferred_element_type=jnp.float32)

    return pl.pallas_call(
        body,
        grid=(n // bn,),
        in_specs=[
            pl.BlockSpec((bn, kdim), lambda i: (i, 0)),
            pl.BlockSpec((bn, 1), lambda i: (i, 0)),
            pl.BlockSpec((kdim, d2), lambda i: (0, 0)),
        ],
        out_specs=pl.BlockSpec((bn, d2), lambda i: (i, 0)),
        out_shape=jax.ShapeDtypeStruct((n, d2), jnp.float32),
    )(x, rs, w)


def _layer_mid(s, rs_in, b, rs_out, w):
    kdim = s.shape[2]
    n = rs_in.shape[0]
    d2 = w.shape[1]
    bn = 2000

    def body(s_ref, ri_ref, b_ref, ro_ref, w_ref, o_ref):
        h = jax.nn.relu((s_ref[0] + s_ref[1]) * ri_ref[...] + b_ref[...])
        o_ref[...] = jnp.dot(h * ro_ref[...], w_ref[...],
                             preferred_element_type=jnp.float32)

    return pl.pallas_call(
        body,
        grid=(n // bn,),
        in_specs=[
            pl.BlockSpec((2, bn, kdim), lambda i: (0, i, 0)),
            pl.BlockSpec((bn, 1), lambda i: (i, 0)),
            pl.BlockSpec((1, kdim), lambda i: (0, 0)),
            pl.BlockSpec((bn, 1), lambda i: (i, 0)),
            pl.BlockSpec((kdim, d2), lambda i: (0, 0)),
        ],
        out_specs=pl.BlockSpec((bn, d2), lambda i: (i, 0)),
        out_shape=jax.ShapeDtypeStruct((n, d2), jnp.float32),
    )(s, rs_in, b, rs_out, w)


def _final(s, rs_in, b):
    d2 = s.shape[2]
    n = rs_in.shape[0]
    bn = 2000

    def body(s_ref, ri_ref, b_ref, o_ref):
        o_ref[...] = (s_ref[0] + s_ref[1]) * ri_ref[...] + b_ref[...]

    return pl.pallas_call(
        body,
        grid=(n // bn,),
        in_specs=[
            pl.BlockSpec((2, bn, d2), lambda i: (0, i, 0)),
            pl.BlockSpec((bn, 1), lambda i: (i, 0)),
            pl.BlockSpec((1, d2), lambda i: (0, 0)),
        ],
        out_specs=pl.BlockSpec((bn, d2), lambda i: (i, 0)),
        out_shape=jax.ShapeDtypeStruct((n, d2), jnp.float32),
    )(s, rs_in, b)


def kernel(x, edge_index, W0, b0, W1, b1, W2, b2):
    n, d_in = x.shape
    e = edge_index.shape[1]
    d_h = W0.shape[1]
    n_cls = W2.shape[1]
    nch = e // NW // CH

    ei4 = jnp.transpose(edge_index.reshape(2, NW, nch, CH), (1, 2, 0, 3))

    o_c0, o_c1, i_c0, i_c1 = _make_deg_kernel(n, e)(ei4)
    rs = _rs_from_deg(o_c0, o_c1, i_c0, i_c1)
    rs_out = rs[0].reshape(n, 1)
    rs_in = rs[1].reshape(n, 1)

    edge128 = _make_edge_kernel(n, e, d_h)
    edge_cls = _make_edge_kernel(n, e, n_cls)

    g = _scale_mm(x, rs_out, W0)
    s = edge128(g, ei4)
    g = _layer_mid(s, rs_in, b0.reshape(1, d_h), rs_out, W1)
    s = edge128(g, ei4)
    g = _layer_mid(s, rs_in, b1.reshape(1, d_h), rs_out, W2)
    s = edge_cls(g, ei4)
    return _final(s, rs_in, b2.reshape(1, n_cls))


# speedup vs baseline: 1.1265x; 1.0650x over previous
import functools

import jax
import jax.numpy as jnp
from jax import lax
from jax.experimental import pallas as pl
from jax.experimental.pallas import tpu as pltpu
from jax.experimental.pallas import tpu_sc as plsc

NC = 2
NS = 16
NW = NC * NS
CH = 80
NP = 10240


def _fill_2d(ref, nrows, ncols, value):
    v = jnp.full((16,), value, jnp.float32)
    npc = ncols // 16

    def body(i, _):
        r = i // npc
        c = (i % npc) * 16
        ref[r, pl.ds(c, 16)] = v
        return 0

    lax.fori_loop(0, nrows * npc, body, 0)


def _fill_1d(ref, n, value):
    v = jnp.full((16,), value, jnp.float32)

    def body(i, _):
        ref[pl.ds(i * 16, 16)] = v
        return 0

    lax.fori_loop(0, n // 16, body, 0)


@functools.lru_cache(maxsize=None)
def _make_deg_kernel(n, e, ch):
    epw = e // NW
    nch = epw // ch
    mesh = plsc.VectorSubcoreMesh(core_axis_name="c", subcore_axis_name="s")
    out1 = jax.ShapeDtypeStruct((n,), jnp.float32)

    @functools.partial(
        pl.kernel,
        out_type=(out1, out1, out1, out1),
        mesh=mesh,
        scratch_types=[
            pltpu.VMEM((n,), jnp.float32),
            pltpu.VMEM((ch,), jnp.float32),
            pltpu.VMEM((10, 2, ch), jnp.int32),
            pltpu.VMEM_SHARED((n,), jnp.float32),
            pltpu.VMEM_SHARED((n,), jnp.float32),
            pltpu.SemaphoreType.DMA,
            pltpu.SemaphoreType.DMA,
        ],
    )
    def k(ei_hbm, o_c0, o_c1, i_c0, i_c1,
          zbuf, ones, idx, acc_o, acc_i, sem_i, sem_s):
        cid = lax.axis_index("c")
        sid = lax.axis_index("s")
        wid = sid * NC + cid

        def idx_load(t):
            pltpu.async_copy(ei_hbm.at[wid, t], idx.at[t % 10], sem_i)

        def wait_idx():
            pltpu.make_async_copy(ei_hbm.at[0, 0], idx.at[0], sem_i).wait()

        def wait_scatter():
            pltpu.make_async_copy(ones, acc_o.at[idx.at[0, 0]], sem_s).wait()

        for t in range(4):
            idx_load(t)
        _fill_1d(ones, ch, 1.0)

        @pl.when(sid == 0)
        def _():
            _fill_1d(zbuf, n, 0.0)
            pltpu.sync_copy(zbuf, acc_o)
            pltpu.sync_copy(zbuf, acc_i)

        plsc.subcore_barrier()

        def body(t, _):
            wait_idx()
            pltpu.async_copy(ones, acc_o.at[idx.at[t % 10, 0]], sem_s, add=True)
            pltpu.async_copy(ones, acc_i.at[idx.at[t % 10, 1]], sem_s, add=True)

            @pl.when(t >= 4)
            def _():
                wait_scatter()
                wait_scatter()

            @pl.when(t + 4 < nch)
            def _():
                idx_load(t + 4)

            return 0

        lax.fori_loop(0, nch, body, 0)
        for _ in range(8):
            wait_scatter()
        plsc.subcore_barrier()

        @pl.when(jnp.logical_and(sid == 0, cid == 0))
        def _():
            pltpu.sync_copy(acc_o, o_c0)
            pltpu.sync_copy(acc_i, i_c0)

        @pl.when(jnp.logical_and(sid == 0, cid == 1))
        def _():
            pltpu.sync_copy(acc_o, o_c1)
            pltpu.sync_copy(acc_i, i_c1)

    return k


@functools.lru_cache(maxsize=None)
def _make_edge_kernel(n, e, d, ch):
    epw = e // NW
    nch = epw // ch
    rpt = NP // NS
    zr = 32
    g_depth = 2 if d >= 128 else 5
    rr = 2 * g_depth
    ri = 2 * g_depth + 2
    mesh = plsc.VectorSubcoreMesh(core_axis_name="c", subcore_axis_name="s")

    @functools.partial(
        pl.kernel,
        out_type=jax.ShapeDtypeStruct((NC, NP, d), jnp.float32),
        mesh=mesh,
        scratch_types=[
            pltpu.VMEM((ri, 2, ch), jnp.int32),
            pltpu.VMEM((rr, ch, d), jnp.float32),
            pltpu.VMEM((zr, d), jnp.float32),
            pltpu.VMEM_SHARED((NP, d), jnp.float32),
            pltpu.SemaphoreType.DMA,
            pltpu.SemaphoreType.DMA,
            pltpu.SemaphoreType.DMA,
        ],
        compiler_params=pltpu.CompilerParams(use_tc_tiling_on_sc=(d % 128 == 0)),
    )
    def k(g_hbm, ei_hbm, out, idx, rows, zbuf, acc, sem_i, sem_g, sem_s):
        cid = lax.axis_index("c")
        sid = lax.axis_index("s")
        wid = sid * NC + cid

        def idx_load(t):
            pltpu.async_copy(ei_hbm.at[wid, t], idx.at[t % ri], sem_i)

        def gather(t):
            pltpu.async_copy(g_hbm.at[idx.at[t % ri, 0]], rows.at[t % rr], sem_g)

        def scatter(t):
            pltpu.async_copy(rows.at[t % rr], acc.at[idx.at[t % ri, 1]], sem_s,
                             add=True)

        def wait_idx():
            pltpu.make_async_copy(ei_hbm.at[0, 0], idx.at[0], sem_i).wait()

        def wait_gather():
            pltpu.make_async_copy(g_hbm.at[idx.at[0, 0]], rows.at[0], sem_g).wait()

        def wait_scatter():
            pltpu.make_async_copy(rows.at[0], acc.at[idx.at[0, 1]], sem_s).wait()

        for t in range(g_depth + 1):
            idx_load(t)
        _fill_2d(zbuf, zr, d, 0.0)
        r0 = sid * rpt
        for j in range(rpt // zr):
            pltpu.async_copy(zbuf, acc.at[pl.ds(r0 + j * zr, zr)], sem_s)
        for j in range(rpt // zr):
            pltpu.make_async_copy(zbuf, acc.at[pl.ds(r0, zr)], sem_s).wait()
        plsc.subcore_barrier()

        for t in range(g_depth):
            wait_idx()
            gather(t)

        def body(t, _):
            wait_gather()
            scatter(t)

            @pl.when(t + g_depth < nch)
            def _():
                wait_idx()

                @pl.when(t >= g_depth)
                def _():
                    wait_scatter()

                gather(t + g_depth)

                @pl.when(t + g_depth + 1 < nch)
                def _():
                    idx_load(t + g_depth + 1)

            return 0

        lax.fori_loop(0, nch, body, 0)
        for _ in range(2 * g_depth):
            wait_scatter()
        plsc.subcore_barrier()
        pltpu.sync_copy(acc.at[pl.ds(r0, rpt)], out.at[cid, pl.ds(r0, rpt)])

    return k


def _rs_from_deg(o_c0, o_c1, i_c0, i_c1):
    def body(a_ref, b_ref, c_ref, d_ref, o_ref):
        o_ref[0, :] = lax.rsqrt(jnp.maximum(a_ref[...] + b_ref[...], 1.0))
        o_ref[1, :] = lax.rsqrt(jnp.maximum(c_ref[...] + d_ref[...], 1.0))

    return pl.pallas_call(
        body,
        out_shape=jax.ShapeDtypeStruct((2, o_c0.shape[0]), jnp.float32),
    )(o_c0, o_c1, i_c0, i_c1)


def _scale_mm(x, rs, w):
    n, kdim = x.shape
    d2 = w.shape[1]
    bn = 2000

    def body(x_ref, rs_ref, w_ref, o_ref):
        o_ref[...] = jnp.dot(x_ref[...] * rs_ref[...], w_ref[...],
                             preferred_element_type=jnp.float32)

    return pl.pallas_call(
        body,
        grid=(n // bn,),
        in_specs=[
            pl.BlockSpec((bn, kdim), lambda i: (i, 0)),
            pl.BlockSpec((bn, 1), lambda i: (i, 0)),
            pl.BlockSpec((kdim, d2), lambda i: (0, 0)),
        ],
        out_specs=pl.BlockSpec((bn, d2), lambda i: (i, 0)),
        out_shape=jax.ShapeDtypeStruct((n, d2), jnp.float32),
    )(x, rs, w)


def _layer_mid(s, rs_in, b, rs_out, w):
    kdim = s.shape[2]
    n = rs_in.shape[0]
    d2 = w.shape[1]
    bn = 2000

    def body(s_ref, ri_ref, b_ref, ro_ref, w_ref, o_ref):
        h = jax.nn.relu((s_ref[0] + s_ref[1]) * ri_ref[...] + b_ref[...])
        o_ref[...] = jnp.dot(h * ro_ref[...], w_ref[...],
                             preferred_element_type=jnp.float32)

    return pl.pallas_call(
        body,
        grid=(n // bn,),
        in_specs=[
            pl.BlockSpec((2, bn, kdim), lambda i: (0, i, 0)),
            pl.BlockSpec((bn, 1), lambda i: (i, 0)),
            pl.BlockSpec((1, kdim), lambda i: (0, 0)),
            pl.BlockSpec((bn, 1), lambda i: (i, 0)),
            pl.BlockSpec((kdim, d2), lambda i: (0, 0)),
        ],
        out_specs=pl.BlockSpec((bn, d2), lambda i: (i, 0)),
        out_shape=jax.ShapeDtypeStruct((n, d2), jnp.float32),
    )(s, rs_in, b, rs_out, w)


def _final(s, rs_in, b):
    d2 = s.shape[2]
    n = rs_in.shape[0]
    bn = 2000

    def body(s_ref, ri_ref, b_ref, o_ref):
        o_ref[...] = (s_ref[0] + s_ref[1]) * ri_ref[...] + b_ref[...]

    return pl.pallas_call(
        body,
        grid=(n // bn,),
        in_specs=[
            pl.BlockSpec((2, bn, d2), lambda i: (0, i, 0)),
            pl.BlockSpec((bn, 1), lambda i: (i, 0)),
            pl.BlockSpec((1, d2), lambda i: (0, 0)),
        ],
        out_specs=pl.BlockSpec((bn, d2), lambda i: (i, 0)),
        out_shape=jax.ShapeDtypeStruct((n, d2), jnp.float32),
    )(s, rs_in, b)


def kernel(x, edge_index, W0, b0, W1, b1, W2, b2):
    n, d_in = x.shape
    e = edge_index.shape[1]
    d_h = W0.shape[1]
    n_cls = W2.shape[1]
    ch_a, ch_b = CH, 125
    nch_a = e // NW // ch_a
    nch_b = e // NW // ch_b
    ei4a = jnp.transpose(edge_index.reshape(2, NW, nch_a, ch_a), (1, 2, 0, 3))
    ei4b = jnp.transpose(edge_index.reshape(2, NW, nch_b, ch_b), (1, 2, 0, 3))

    o_c0, o_c1, i_c0, i_c1 = _make_deg_kernel(n, e, ch_b)(ei4b)
    rs = _rs_from_deg(o_c0, o_c1, i_c0, i_c1)
    rs_out = rs[0].reshape(n, 1)
    rs_in = rs[1].reshape(n, 1)

    edge128 = _make_edge_kernel(n, e, d_h, ch_a)
    edge_cls = _make_edge_kernel(n, e, n_cls, ch_b)

    g = _scale_mm(x, rs_out, W0)
    s = edge128(g, ei4a)
    g = _layer_mid(s, rs_in, b0.reshape(1, d_h), rs_out, W1)
    s = edge128(g, ei4a)
    g = _layer_mid(s, rs_in, b1.reshape(1, d_h), rs_out, W2)
    s = edge_cls(g, ei4b)
    return _final(s, rs_in, b2.reshape(1, n_cls))
